# baseline (speedup 1.0000x reference)
import functools

import jax
import jax.numpy as jnp
from jax.experimental import pallas as pl
from jax.experimental.pallas import tpu as pltpu
from jax.experimental.pallas import tpu_sc as plsc

E, H, N, HD, B, K = 8, 16, 512, 64, 16, 2
G = 4
NB = 14
NBP = 16
P = NBP * G
PAIRS = B * K


def _routing_sc(idx_t, ew_t):
    hh = H // 2

    mesh = plsc.ScalarSubcoreMesh(axis_name="core", num_cores=2)

    @functools.partial(
        pl.kernel,
        out_type=(
            jax.ShapeDtypeStruct((H * P,), jnp.int32),
            jax.ShapeDtypeStruct((H * P,), jnp.float32),
            jax.ShapeDtypeStruct((H * NBP,), jnp.int32),
        ),
        mesh=mesh,
        scratch_types=[
            pltpu.SMEM((hh * PAIRS,), jnp.int32),
            pltpu.SMEM((hh * PAIRS,), jnp.float32),
            pltpu.SMEM((hh * P,), jnp.int32),
            pltpu.SMEM((hh * P,), jnp.float32),
            pltpu.SMEM((hh * NBP,), jnp.int32),
            pltpu.SMEM((E,), jnp.int32),
            pltpu.SMEM((E,), jnp.int32),
            pltpu.SMEM((E,), jnp.int32),
            pltpu.SMEM((2,), jnp.int32),
            pltpu.SemaphoreType.DMA,
        ],
    )
    def sc_kernel(idx_hbm, ew_hbm, pb_hbm, pw_hbm, be_hbm,
                  idx_s, ew_s, pb_s, pw_s, be_s,
                  cnt_s, off_s, fill_s, cur_s, sem):
        core = jax.lax.axis_index("core")
        pltpu.async_copy(idx_hbm.at[pl.ds(core * hh * PAIRS, hh * PAIRS)],
                         idx_s, sem).wait()
        pltpu.async_copy(ew_hbm.at[pl.ds(core * hh * PAIRS, hh * PAIRS)],
                         ew_s, sem).wait()

        @pl.loop(0, hh)
        def _(hl):
            in0 = hl * PAIRS

            @pl.loop(0, E)
            def _(e):
                cnt_s[e] = 0
                fill_s[e] = 0

            @pl.loop(0, PAIRS)
            def _(i):
                cnt_s[idx_s[in0 + i]] += 1

            cur_s[0] = 0
            cur_s[1] = 0

            @pl.loop(0, E)
            def _(e):
                n = cnt_s[e]

                @pl.when(n > 0)
                def _():
                    off_s[e] = cur_s[0]
                    nb_e = (n + (G - 1)) // G

                    @pl.loop(0, NB)
                    def _(i):
                        @pl.when(i < nb_e)
                        def _():
                            be_s[hl * NBP + cur_s[1] + i] = e

                    cur_s[0] += nb_e * G
                    cur_s[1] += nb_e

            last_e = be_s[hl * NBP + cur_s[1] - 1]

            @pl.loop(0, NBP)
            def _(i):
                @pl.when(i >= cur_s[1])
                def _():
                    be_s[hl * NBP + i] = last_e

            @pl.loop(0, P)
            def _(i):
                pb_s[hl * P + i] = 0
                pw_s[hl * P + i] = 0.0

            @pl.loop(0, PAIRS)
            def _(i):
                e = idx_s[in0 + i]
                slot = off_s[e] + fill_s[e]
                fill_s[e] += 1
                pb_s[hl * P + slot] = i // K
                pw_s[hl * P + slot] = ew_s[in0 + i]

        pltpu.async_copy(pb_s, pb_hbm.at[pl.ds(core * hh * P, hh * P)],
                         sem).wait()
        pltpu.async_copy(pw_s, pw_hbm.at[pl.ds(core * hh * P, hh * P)],
                         sem).wait()
        pltpu.async_copy(be_s, be_hbm.at[pl.ds(core * hh * NBP, hh * NBP)],
                         sem).wait()

    return sc_kernel(idx_t, ew_t)


def _mix_tc_body(be_ref, pb_ref, coef_ref, bias_ref, w_ref, x_ref, out_ref,
                 xb_ref, s_ref, invr_ref):
    h = pl.program_id(0)
    j = pl.program_id(1)

    @pl.when(j == 0)
    def _():
        xb_ref[...] = x_ref[...].reshape(B * HD, N).astype(jnp.bfloat16)
        out_ref[...] = jnp.zeros_like(out_ref)

    cur_e = be_ref[h, j]
    prev_e = be_ref[h, jnp.maximum(j - 1, 0)]

    @pl.when((j == 0) | (cur_e != prev_e))
    def _():
        w = w_ref[cur_e, 0]
        ew_mat = jnp.exp(w)
        s_ref[...] = ew_mat.astype(jnp.bfloat16)
        r = jnp.sum(ew_mat, axis=1, keepdims=True)
        invr_ref[...] = (1.0 / r).reshape(1, N)

    xg = jnp.concatenate(
        [xb_ref[pl.ds(pb_ref[h, G * j + i] * HD, HD), :] for i in range(G)],
        axis=0)
    y = jax.lax.dot_general(xg, s_ref[...], (((1,), (1,)), ((), ())),
                            preferred_element_type=jnp.float32)
    contrib = (y * invr_ref[...] + bias_ref[cur_e, 0]) * coef_ref[0]

    for i in range(G):
        b_i = pb_ref[h, G * j + i]
        sl = (pl.ds(b_i, 1), slice(None), slice(None), slice(None))
        out_ref[sl] = out_ref[sl] + contrib[i * HD:(i + 1) * HD, :].reshape(1, 1, HD, N)


def kernel(x, expert_indices, expert_weights, weight, bias):
    idx_t = jnp.transpose(expert_indices.astype(jnp.int32), (1, 0, 2)).reshape(-1)
    ew_t = jnp.transpose(expert_weights, (1, 0, 2)).reshape(-1)

    pair_b, pair_w, blk_e = _routing_sc(idx_t, ew_t)
    pair_b = pair_b.reshape(H, P)
    blk_e = blk_e.reshape(H, NBP)
    coef = jnp.broadcast_to(pair_w.reshape(H, P)[:, :, None],
                            (H, P, HD)).reshape(H, P * HD, 1)
    bias_r = bias.reshape(E, H, 1, N)

    grid_spec = pltpu.PrefetchScalarGridSpec(
        num_scalar_prefetch=2,
        grid=(H, NB),
        in_specs=[
            pl.BlockSpec((1, G * HD, 1),
                         lambda h, j, be, pb: (h, j, 0)),
            pl.BlockSpec((E, 1, 1, N),
                         lambda h, j, be, pb: (0, h, 0, 0)),
            pl.BlockSpec((E, 1, N, N),
                         lambda h, j, be, pb: (0, h, 0, 0)),
            pl.BlockSpec((B, 1, HD, N),
                         lambda h, j, be, pb: (0, h, 0, 0)),
        ],
        out_specs=pl.BlockSpec((B, 1, HD, N),
                               lambda h, j, be, pb: (0, h, 0, 0)),
        scratch_shapes=[
            pltpu.VMEM((B * HD, N), jnp.bfloat16),
            pltpu.VMEM((N, N), jnp.bfloat16),
            pltpu.VMEM((1, N), jnp.float32),
        ],
    )
    out = pl.pallas_call(
        _mix_tc_body,
        grid_spec=grid_spec,
        out_shape=jax.ShapeDtypeStruct((B, H, HD, N), jnp.float32),
        compiler_params=pltpu.CompilerParams(
            dimension_semantics=("arbitrary", "arbitrary"),
        ),
    )(blk_e, pair_b, coef, bias_r, weight, x)
    return out


# speedup vs baseline: 1.6972x; 1.6972x over previous
import functools

import jax
import jax.numpy as jnp
from jax.experimental import pallas as pl
from jax.experimental.pallas import tpu as pltpu
from jax.experimental.pallas import tpu_sc as plsc

E, H, N, HD, B, K = 8, 16, 512, 64, 16, 2


def _routing_coeffs_sc(idx_flat, ew_flat):
    n_pairs = B * H * K
    n_rows = B * H
    half_pairs = n_pairs // 2
    half_rows = n_rows // 2

    mesh = plsc.ScalarSubcoreMesh(axis_name="core", num_cores=2)

    @functools.partial(
        pl.kernel,
        out_type=jax.ShapeDtypeStruct((n_rows * E,), jnp.float32),
        mesh=mesh,
        scratch_types=[
            pltpu.SMEM((half_pairs,), jnp.int32),
            pltpu.SMEM((half_pairs,), jnp.float32),
            pltpu.SMEM((half_rows * E,), jnp.float32),
            pltpu.SemaphoreType.DMA,
        ],
    )
    def sc_kernel(idx_hbm, ew_hbm, out_hbm, idx_s, ew_s, acc_s, sem):
        core = jax.lax.axis_index("core")
        pltpu.async_copy(idx_hbm.at[pl.ds(core * half_pairs, half_pairs)],
                         idx_s, sem).wait()
        pltpu.async_copy(ew_hbm.at[pl.ds(core * half_pairs, half_pairs)],
                         ew_s, sem).wait()

        @pl.loop(0, half_rows * E)
        def _(i):
            acc_s[i] = 0.0

        @pl.loop(0, half_pairs)
        def _(i):
            row_local = i // K
            e = idx_s[i]
            acc_s[row_local * E + e] += ew_s[i]

        pltpu.async_copy(acc_s,
                         out_hbm.at[pl.ds(core * half_rows * E, half_rows * E)],
                         sem).wait()

    return sc_kernel(idx_flat, ew_flat)


def _mix_tc_body(coef_ref, bias_ref, w_ref, x_ref, out_ref):
    xb = x_ref[...].reshape(B * HD, N).astype(jnp.bfloat16)
    acc = None
    for e in range(E):
        w = w_ref[e, 0]
        ew_mat = jnp.exp(w)
        r = jnp.sum(ew_mat, axis=1, keepdims=True)
        s = (ew_mat / r).astype(jnp.bfloat16)
        y = jax.lax.dot_general(xb, s, (((1,), (1,)), ((), ())),
                                preferred_element_type=jnp.float32)
        term = (y + bias_ref[e, 0]) * coef_ref[0, e]
        acc = term if acc is None else acc + term
    out_ref[...] = acc.reshape(B, 1, HD, N)


def kernel(x, expert_indices, expert_weights, weight, bias):
    idx_flat = expert_indices.astype(jnp.int32).reshape(-1)
    ew_flat = expert_weights.reshape(-1)

    c_flat = _routing_coeffs_sc(idx_flat, ew_flat)
    c = c_flat.reshape(B, H, E)
    coef = jnp.broadcast_to(
        jnp.transpose(c, (1, 2, 0))[:, :, :, None, None],
        (H, E, B, HD, 1),
    ).reshape(H, E, B * HD, 1)
    bias_r = bias.reshape(E, H, 1, N)

    out = pl.pallas_call(
        _mix_tc_body,
        grid=(H,),
        in_specs=[
            pl.BlockSpec((1, E, B * HD, 1), lambda h: (h, 0, 0, 0)),
            pl.BlockSpec((E, 1, 1, N), lambda h: (0, h, 0, 0)),
            pl.BlockSpec((E, 1, N, N), lambda h: (0, h, 0, 0)),
            pl.BlockSpec((B, 1, HD, N), lambda h: (0, h, 0, 0)),
        ],
        out_specs=pl.BlockSpec((B, 1, HD, N), lambda h: (0, h, 0, 0)),
        out_shape=jax.ShapeDtypeStruct((B, H, HD, N), jnp.float32),
        compiler_params=pltpu.CompilerParams(
            dimension_semantics=("arbitrary",),
        ),
    )(coef, bias_r, weight, x)
    return out
